# Initial kernel scaffold; baseline (speedup 1.0000x reference)
import jax
import jax.numpy as jnp
from jax.experimental import pallas as pl


def kernel(token_ids, embedding):
    raise NotImplementedError("write your pallas kernel here")


# speedup vs baseline: 4.9464x; 4.9464x over previous
import functools

import jax
import jax.numpy as jnp
from jax import lax
from jax.experimental import pallas as pl
from jax.experimental.pallas import tpu as pltpu
from jax.experimental.pallas import tpu_sc as plsc

D = 32
NW = 32
G = 128
K = 16
STEP = K * G


@functools.lru_cache(maxsize=None)
def _make_gather(B):
    assert B % (NW * STEP) == 0
    steps = B // (NW * STEP)
    mesh = plsc.VectorSubcoreMesh(core_axis_name="c", subcore_axis_name="s")

    @functools.partial(
        pl.kernel,
        mesh=mesh,
        out_type=jax.ShapeDtypeStruct((B, D), jnp.float32),
        scratch_types=[
            pltpu.VMEM((K, G), jnp.int32),
            pltpu.VMEM((STEP, D), jnp.float32),
            pltpu.SemaphoreType.DMA,
        ],
        compiler_params=pltpu.CompilerParams(use_tc_tiling_on_sc=False),
    )
    def gather(idx_hbm, table_hbm, out_hbm, idx_v, rows_v, sem):
        wid = lax.axis_index("s") * 2 + lax.axis_index("c")
        base = wid * steps

        def body(i, carry):
            blk = base + i
            pltpu.sync_copy(idx_hbm.at[blk], idx_v)
            copies = [
                pltpu.async_copy(
                    table_hbm.at[idx_v.at[j]],
                    rows_v.at[pl.ds(j * G, G)],
                    sem,
                )
                for j in range(K)
            ]
            for c in copies:
                c.wait()
            pltpu.sync_copy(rows_v, out_hbm.at[pl.ds(blk * STEP, STEP)])
            return carry

        lax.fori_loop(0, steps, body, 0)

    return gather


def kernel(token_ids, embedding):
    B = token_ids.size
    idx = token_ids.reshape(B // STEP, K, G).astype(jnp.int32)
    out = _make_gather(B)(idx, embedding)
    return out.reshape(token_ids.shape + (D,))


# speedup vs baseline: 4.9739x; 1.0056x over previous
import functools

import jax
import jax.numpy as jnp
from jax import lax
from jax.experimental import pallas as pl
from jax.experimental.pallas import tpu as pltpu
from jax.experimental.pallas import tpu_sc as plsc

D = 32
NW = 32
G = 128
K = 10
STEP = K * G


@functools.lru_cache(maxsize=None)
def _make_gather(B):
    assert B % (NW * STEP) == 0
    steps = B // (NW * STEP)
    assert steps % 2 == 0
    mesh = plsc.VectorSubcoreMesh(core_axis_name="c", subcore_axis_name="s")

    @functools.partial(
        pl.kernel,
        mesh=mesh,
        out_type=jax.ShapeDtypeStruct((B, D), jnp.float32),
        scratch_types=[
            pltpu.VMEM((2 * K, G), jnp.int32),
            pltpu.VMEM((2 * STEP, D), jnp.float32),
            pltpu.SemaphoreType.DMA,
            pltpu.SemaphoreType.DMA,
        ],
        compiler_params=pltpu.CompilerParams(use_tc_tiling_on_sc=False),
    )
    def gather(idx_hbm, table_hbm, out_hbm, idx_v, rows_v, sem0, sem1):
        wid = lax.axis_index("s") * 2 + lax.axis_index("c")
        base = wid * steps
        sems = (sem0, sem1)

        def fire(chunk, b):
            pltpu.sync_copy(idx_hbm.at[chunk], idx_v.at[pl.ds(b * K, K)])
            for j in range(K):
                pltpu.async_copy(
                    table_hbm.at[idx_v.at[b * K + j]],
                    rows_v.at[pl.ds(b * STEP + j * G, G)],
                    sems[b],
                )

        def drain_and_write(chunk, b):
            pltpu.make_async_copy(
                out_hbm.at[pl.ds(0, STEP)],
                rows_v.at[pl.ds(b * STEP, STEP)],
                sems[b],
            ).wait()
            pltpu.sync_copy(
                rows_v.at[pl.ds(b * STEP, STEP)],
                out_hbm.at[pl.ds(chunk * STEP, STEP)],
            )

        def half(i, b):
            @pl.when(i + 1 < steps)
            def _():
                fire(base + i + 1, 1 - b)

            drain_and_write(base + i, b)

        fire(base, 0)

        def body(t, carry):
            half(2 * t, 0)
            half(2 * t + 1, 1)
            return carry

        lax.fori_loop(0, steps // 2, body, 0)

    return gather


def kernel(token_ids, embedding):
    B = token_ids.size
    idx = token_ids.reshape(B // STEP, K, G).astype(jnp.int32)
    out = _make_gather(B)(idx, embedding)
    return out.reshape(token_ids.shape + (D,))


# speedup vs baseline: 5.0618x; 1.0177x over previous
import functools

import jax
import jax.numpy as jnp
from jax import lax
from jax.experimental import pallas as pl
from jax.experimental.pallas import tpu as pltpu
from jax.experimental.pallas import tpu_sc as plsc

D = 32
PITCH = 33
NT = 25
NC = 128
CPW = 4
NBLK = NT * CPW
NU = 8 * NBLK

_MESH = dict(core_axis_name="c", subcore_axis_name="s")


@functools.lru_cache(maxsize=None)
def _make_gather(V):
    @functools.partial(
        pl.kernel,
        mesh=plsc.VectorSubcoreMesh(**_MESH),
        out_type=jax.ShapeDtypeStruct((NT * 8 * NC, 128, D), jnp.float32),
        scratch_types=[
            pltpu.VMEM((16, 128), jnp.int32),
            pltpu.VMEM((16, 128, D), jnp.float32),
            pltpu.SemaphoreType.DMA,
            pltpu.SemaphoreType.DMA,
            pltpu.SemaphoreType.DMA,
        ],
        compiler_params=pltpu.CompilerParams(use_tc_tiling_on_sc=False),
    )
    def gather(idx_hbm, table_hbm, mid_hbm, idx_v, rows_v, g0, g1, wsem):
        w = lax.axis_index("s") * 2 + lax.axis_index("c")
        gsems = (g0, g1)

        def gather_copy(b, j):
            return pltpu.make_async_copy(
                table_hbm.at[idx_v.at[b * 8 + j]],
                rows_v.at[b * 8 + j],
                gsems[b],
            )

        def write_copy(blk, b, j):
            T = blk // CPW
            C = w * CPW + blk % CPW
            return pltpu.make_async_copy(
                rows_v.at[b * 8 + j],
                mid_hbm.at[1024 * T + 128 * j + C],
                wsem,
            )

        def fire(blk, b):
            T = blk // CPW
            C = w * CPW + blk % CPW

            @pl.when(blk >= 2)
            def _():
                for j in range(8):
                    write_copy(blk - 2, b, j).wait()

            pltpu.sync_copy(idx_hbm.at[T, C], idx_v.at[pl.ds(b * 8, 8)])
            for j in range(8):
                gather_copy(b, j).start()

        def process(blk, b):
            for j in range(8):
                gather_copy(b, j).wait()
                write_copy(blk, b, j).start()

        fire(0, 0)

        def body(tau, carry):
            i0 = 2 * tau

            @pl.when(i0 + 1 < NBLK)
            def _():
                fire(i0 + 1, 1)

            process(i0, 0)

            @pl.when(i0 + 2 < NBLK)
            def _():
                fire(i0 + 2, 0)

            process(i0 + 1, 1)
            return carry

        lax.fori_loop(0, NBLK // 2, body, 0)
        for j in range(8):
            write_copy(NBLK - 2, 0, j).wait()
            write_copy(NBLK - 1, 1, j).wait()

    return gather


@functools.lru_cache(maxsize=None)
def _make_transpose():
    @functools.partial(
        pl.kernel,
        mesh=plsc.VectorSubcoreMesh(**_MESH),
        out_type=jax.ShapeDtypeStruct((NT * 32, NC, 8, 128), jnp.float32),
        scratch_types=[
            pltpu.VMEM((128, PITCH), jnp.float32),
            pltpu.VMEM((128, PITCH), jnp.float32),
            pltpu.VMEM((4, 8, 128), jnp.float32),
            pltpu.VMEM((4, 8, 128), jnp.float32),
            pltpu.SemaphoreType.DMA,
            pltpu.SemaphoreType.DMA,
            pltpu.SemaphoreType.DMA,
            pltpu.SemaphoreType.DMA,
        ],
        compiler_params=pltpu.CompilerParams(
            use_tc_tiling_on_sc=False, needs_layout_passes=False
        ),
    )
    def transpose(mid_hbm, out_hbm, rows0, rows1, tiles0, tiles1,
                  r0, r1, w0, w1):
        w = lax.axis_index("s") * 2 + lax.axis_index("c")
        rows = (rows0, rows1)
        tiles = (tiles0, tiles1)
        rsems = (r0, r1)
        wsems = (w0, w1)
        iota = lax.iota(jnp.int32, 16)
        c_idx = [g * 16 + iota for g in range(8)]

        def coords(u):
            blk = u // 8
            j = u % 8
            T = blk // CPW
            C = w * CPW + blk % CPW
            return T, C, j

        def read_copy(u, b):
            T, C, j = coords(u)
            return pltpu.make_async_copy(
                mid_hbm.at[1024 * T + 128 * j + C],
                rows[b].at[:, pl.ds(0, D)],
                rsems[b],
            )

        def tile_copy(u, b, k):
            T, C, j = coords(u)
            return pltpu.make_async_copy(
                tiles[b].at[k],
                out_hbm.at[32 * T + 4 * j + k, C],
                wsems[b],
            )

        def process(u, b):
            read_copy(u, b).wait()

            @pl.when(u >= 2)
            def _():
                for k in range(4):
                    tile_copy(u - 2, b, k).wait()

            src = rows[b]
            dst = tiles[b]
            for k in range(4):

                def tbody(r2, carry, _k=k):
                    d_vec = jnp.full((16,), _k * 8 + r2, dtype=jnp.int32)
                    for g in range(8):
                        val = plsc.load_gather(src, [c_idx[g], d_vec])
                        dst[_k, r2, pl.ds(g * 16, 16)] = val
                    return carry

                lax.fori_loop(0, 8, tbody, 0)
            for k in range(4):
                tile_copy(u, b, k).start()

        read_copy(0, 0).start()

        def body(tau, carry):
            u0 = 2 * tau

            @pl.when(u0 + 1 < NU)
            def _():
                read_copy(u0 + 1, 1).start()

            process(u0, 0)

            @pl.when(u0 + 2 < NU)
            def _():
                read_copy(u0 + 2, 0).start()

            process(u0 + 1, 1)
            return carry

        lax.fori_loop(0, NU // 2, body, 0)
        for k in range(4):
            tile_copy(NU - 2, 0, k).wait()
            tile_copy(NU - 1, 1, k).wait()

    return transpose


def kernel(token_ids, embedding):
    NB, NS = token_ids.shape
    idx4 = (
        token_ids.astype(jnp.int32)
        .reshape(NC, 128, NT, 8)
        .transpose(2, 0, 3, 1)
    )
    mid = _make_gather(embedding.shape[0])(idx4, embedding)
    out4 = _make_transpose()(mid)
    out = (
        out4.reshape(NS, 4, NC, 8, 128)
        .transpose(2, 4, 0, 1, 3)
        .reshape(NB, NS, D)
    )
    return out


# speedup vs baseline: 7.4482x; 1.4715x over previous
import functools

import jax
import jax.numpy as jnp
from jax import lax
from jax.experimental import pallas as pl
from jax.experimental.pallas import tpu as pltpu
from jax.experimental.pallas import tpu_sc as plsc

D = 32
PITCH = 33
NT = 25
NC = 128
CPW = 4
NBLK = NT * CPW
NU = 8 * NBLK

_MESH = dict(core_axis_name="c", subcore_axis_name="s")


@functools.lru_cache(maxsize=None)
def _make_gather(V):
    @functools.partial(
        pl.kernel,
        mesh=plsc.VectorSubcoreMesh(**_MESH),
        out_type=jax.ShapeDtypeStruct((NT * 8 * NC, 128, D), jnp.float32),
        scratch_types=[
            pltpu.VMEM((16, 128), jnp.int32),
            pltpu.VMEM((16, 128, D), jnp.float32),
            pltpu.SemaphoreType.DMA,
            pltpu.SemaphoreType.DMA,
            pltpu.SemaphoreType.DMA,
        ],
        compiler_params=pltpu.CompilerParams(use_tc_tiling_on_sc=False),
    )
    def gather(idx_hbm, table_hbm, mid_hbm, idx_v, rows_v, g0, g1, wsem):
        w = lax.axis_index("s") * 2 + lax.axis_index("c")
        gsems = (g0, g1)

        def gather_copy(b, j):
            return pltpu.make_async_copy(
                table_hbm.at[idx_v.at[b * 8 + j]],
                rows_v.at[b * 8 + j],
                gsems[b],
            )

        def write_copy(blk, b, j):
            T = blk // CPW
            C = w * CPW + blk % CPW
            return pltpu.make_async_copy(
                rows_v.at[b * 8 + j],
                mid_hbm.at[1024 * T + 128 * j + C],
                wsem,
            )

        def fire(blk, b):
            T = blk // CPW
            C = w * CPW + blk % CPW

            @pl.when(blk >= 2)
            def _():
                for j in range(8):
                    write_copy(blk - 2, b, j).wait()

            pltpu.sync_copy(idx_hbm.at[T, C], idx_v.at[pl.ds(b * 8, 8)])
            for j in range(8):
                gather_copy(b, j).start()

        def process(blk, b):
            for j in range(8):
                gather_copy(b, j).wait()
                write_copy(blk, b, j).start()

        fire(0, 0)

        def body(tau, carry):
            i0 = 2 * tau

            @pl.when(i0 + 1 < NBLK)
            def _():
                fire(i0 + 1, 1)

            process(i0, 0)

            @pl.when(i0 + 2 < NBLK)
            def _():
                fire(i0 + 2, 0)

            process(i0 + 1, 1)
            return carry

        lax.fori_loop(0, NBLK // 2, body, 0)
        for j in range(8):
            write_copy(NBLK - 2, 0, j).wait()
            write_copy(NBLK - 1, 1, j).wait()

    return gather


@functools.lru_cache(maxsize=None)
def _make_transpose():
    @functools.partial(
        pl.kernel,
        mesh=plsc.VectorSubcoreMesh(**_MESH),
        out_type=jax.ShapeDtypeStruct((NT * 32, NC, 8, 128), jnp.float32),
        scratch_types=[
            pltpu.VMEM((128, PITCH), jnp.float32),
            pltpu.VMEM((128, PITCH), jnp.float32),
            pltpu.VMEM((32, 128), jnp.float32),
            pltpu.VMEM((32, 128), jnp.float32),
            pltpu.SemaphoreType.DMA,
            pltpu.SemaphoreType.DMA,
            pltpu.SemaphoreType.DMA,
            pltpu.SemaphoreType.DMA,
        ],
        compiler_params=pltpu.CompilerParams(
            use_tc_tiling_on_sc=False, needs_layout_passes=False
        ),
    )
    def transpose(mid_hbm, out_hbm, rows0, rows1, tiles0, tiles1,
                  r0, r1, w0, w1):
        w = lax.axis_index("s") * 2 + lax.axis_index("c")
        rows = (rows0, rows1)
        tiles = (tiles0, tiles1)
        rsems = (r0, r1)
        wsems = (w0, w1)
        iota = lax.iota(jnp.int32, 16)
        c_idx = [g * 16 + iota for g in range(8)]

        def coords(u):
            blk = u // 8
            j = u % 8
            T = blk // CPW
            C = w * CPW + blk % CPW
            return T, C, j

        def read_copy(u, b):
            T, C, j = coords(u)
            return pltpu.make_async_copy(
                mid_hbm.at[1024 * T + 128 * j + C],
                rows[b].at[:, pl.ds(0, D)],
                rsems[b],
            )

        def tile_copy(u, b, k):
            T, C, j = coords(u)
            return pltpu.make_async_copy(
                tiles[b].at[pl.ds(8 * k, 8)],
                out_hbm.at[32 * T + 4 * j + k, C],
                wsems[b],
            )

        def process(u, b):
            read_copy(u, b).wait()

            @pl.when(u >= 2)
            def _():
                for k in range(4):
                    tile_copy(u - 2, b, k).wait()

            src = rows[b]
            dst = tiles[b]

            @plsc.parallel_loop(0, 32, unroll=8)
            def _t(d):
                d_vec = jnp.full((16,), d, dtype=jnp.int32)
                for g in range(8):
                    val = plsc.load_gather(src, [c_idx[g], d_vec])
                    dst[d, pl.ds(g * 16, 16)] = val

            for k in range(4):
                tile_copy(u, b, k).start()

        read_copy(0, 0).start()

        def body(tau, carry):
            u0 = 2 * tau

            @pl.when(u0 + 1 < NU)
            def _():
                read_copy(u0 + 1, 1).start()

            process(u0, 0)

            @pl.when(u0 + 2 < NU)
            def _():
                read_copy(u0 + 2, 0).start()

            process(u0 + 1, 1)
            return carry

        lax.fori_loop(0, NU // 2, body, 0)
        for k in range(4):
            tile_copy(NU - 2, 0, k).wait()
            tile_copy(NU - 1, 1, k).wait()

    return transpose


def kernel(token_ids, embedding):
    NB, NS = token_ids.shape
    idx4 = (
        token_ids.astype(jnp.int32)
        .reshape(NC, 128, NT, 8)
        .transpose(2, 0, 3, 1)
    )
    mid = _make_gather(embedding.shape[0])(idx4, embedding)
    out4 = _make_transpose()(mid)
    out = (
        out4.reshape(NS, 4, NC, 8, 128)
        .transpose(2, 4, 0, 1, 3)
        .reshape(NB, NS, D)
    )
    return out


# speedup vs baseline: 7.8407x; 1.0527x over previous
import functools

import jax
import jax.numpy as jnp
from jax import lax
from jax.experimental import pallas as pl
from jax.experimental.pallas import tpu as pltpu
from jax.experimental.pallas import tpu_sc as plsc

D = 32
PITCH = 49
NT = 25
NC = 128
CPW = 4
NBLK = NT * CPW
NU = 8 * NBLK

_MESH = dict(core_axis_name="c", subcore_axis_name="s")


@functools.lru_cache(maxsize=None)
def _make_gather(V):
    @functools.partial(
        pl.kernel,
        mesh=plsc.VectorSubcoreMesh(**_MESH),
        out_type=jax.ShapeDtypeStruct((NT * 8 * NC, 128, D), jnp.float32),
        scratch_types=[
            pltpu.VMEM((16, 128), jnp.int32),
            pltpu.VMEM((16, 128, D), jnp.float32),
            pltpu.SemaphoreType.DMA,
            pltpu.SemaphoreType.DMA,
            pltpu.SemaphoreType.DMA,
        ],
        compiler_params=pltpu.CompilerParams(use_tc_tiling_on_sc=False),
    )
    def gather(idx_hbm, table_hbm, mid_hbm, idx_v, rows_v, g0, g1, wsem):
        w = lax.axis_index("s") * 2 + lax.axis_index("c")
        gsems = (g0, g1)

        def gather_copy(b, j):
            return pltpu.make_async_copy(
                table_hbm.at[idx_v.at[b * 8 + j]],
                rows_v.at[b * 8 + j],
                gsems[b],
            )

        def write_copy(blk, b, j):
            return pltpu.make_async_copy(
                rows_v.at[b * 8 + j],
                mid_hbm.at[(w * NBLK + blk) * 8 + j],
                wsem,
            )

        def fire(blk, b):
            T = blk // CPW
            C = w * CPW + blk % CPW

            @pl.when(blk >= 2)
            def _():
                for j in range(8):
                    write_copy(blk - 2, b, j).wait()

            pltpu.sync_copy(idx_hbm.at[T, C], idx_v.at[pl.ds(b * 8, 8)])
            for j in range(8):
                gather_copy(b, j).start()

        def process(blk, b):
            for j in range(8):
                gather_copy(b, j).wait()
                write_copy(blk, b, j).start()

        fire(0, 0)

        def body(tau, carry):
            i0 = 2 * tau

            @pl.when(i0 + 1 < NBLK)
            def _():
                fire(i0 + 1, 1)

            process(i0, 0)

            @pl.when(i0 + 2 < NBLK)
            def _():
                fire(i0 + 2, 0)

            process(i0 + 1, 1)
            return carry

        lax.fori_loop(0, NBLK // 2, body, 0)
        for j in range(8):
            write_copy(NBLK - 2, 0, j).wait()
            write_copy(NBLK - 1, 1, j).wait()

    return gather


@functools.lru_cache(maxsize=None)
def _make_transpose():
    @functools.partial(
        pl.kernel,
        mesh=plsc.VectorSubcoreMesh(**_MESH),
        out_type=jax.ShapeDtypeStruct((NT * 32, NC, 8, 128), jnp.float32),
        scratch_types=[
            pltpu.VMEM((8, 128, PITCH), jnp.float32),
            pltpu.VMEM((8, 128, PITCH), jnp.float32),
            pltpu.VMEM((32, 128), jnp.float32),
            pltpu.VMEM((32, 128), jnp.float32),
            pltpu.SemaphoreType.DMA,
            pltpu.SemaphoreType.DMA,
            pltpu.SemaphoreType.DMA,
            pltpu.SemaphoreType.DMA,
        ],
        compiler_params=pltpu.CompilerParams(
            use_tc_tiling_on_sc=False, needs_layout_passes=False
        ),
    )
    def transpose(mid_hbm, out_hbm, rows0, rows1, tiles0, tiles1,
                  r0, r1, w0, w1):
        w = lax.axis_index("s") * 2 + lax.axis_index("c")
        rows = (rows0, rows1)
        tiles = (tiles0, tiles1)
        rsems = (r0, r1)
        wsems = (w0, w1)
        iota = lax.iota(jnp.int32, 16)
        c_idx = [g * 16 + iota for g in range(8)]
        j_vec = [jnp.full((16,), j, dtype=jnp.int32) for j in range(8)]

        def read_copy(blk, b):
            return pltpu.make_async_copy(
                mid_hbm.at[pl.ds((w * NBLK + blk) * 8, 8)],
                rows[b].at[:, :, pl.ds(0, D)],
                rsems[b],
            )

        def tile_copy(blk, j, k):
            T = blk // CPW
            C = w * CPW + blk % CPW
            return pltpu.make_async_copy(
                tiles[j % 2].at[pl.ds(8 * k, 8)],
                out_hbm.at[32 * T + 4 * j + k, C],
                wsems[j % 2],
            )

        def process(blk, b):
            read_copy(blk, b).wait()
            for j in range(8):
                if j >= 2:
                    for k in range(4):
                        tile_copy(blk, j - 2, k).wait()
                else:

                    @pl.when(blk > 0)
                    def _(j=j):
                        for k in range(4):
                            tile_copy(blk - 1, j + 6, k).wait()

                src = rows[b]
                dst = tiles[j % 2]

                @plsc.parallel_loop(0, 32, unroll=4)
                def _t(d):
                    d_vec = jnp.full((16,), d, dtype=jnp.int32)
                    for g in range(8):
                        val = plsc.load_gather(
                            src, [j_vec[j], c_idx[g], d_vec]
                        )
                        dst[d, pl.ds(g * 16, 16)] = val

                for k in range(4):
                    tile_copy(blk, j, k).start()

        read_copy(0, 0).start()

        def body(tau, carry):
            i0 = 2 * tau

            @pl.when(i0 + 1 < NBLK)
            def _():
                read_copy(i0 + 1, 1).start()

            process(i0, 0)

            @pl.when(i0 + 2 < NBLK)
            def _():
                read_copy(i0 + 2, 0).start()

            process(i0 + 1, 1)
            return carry

        lax.fori_loop(0, NBLK // 2, body, 0)
        for k in range(4):
            tile_copy(NBLK - 1, 6, k).wait()
            tile_copy(NBLK - 1, 7, k).wait()

    return transpose


def kernel(token_ids, embedding):
    NB, NS = token_ids.shape
    idx4 = (
        token_ids.astype(jnp.int32)
        .reshape(NC, 128, NT, 8)
        .transpose(2, 0, 3, 1)
    )
    mid = _make_gather(embedding.shape[0])(idx4, embedding)
    out4 = _make_transpose()(mid)
    out = (
        out4.reshape(NS, 4, NC, 8, 128)
        .transpose(2, 4, 0, 1, 3)
        .reshape(NB, NS, D)
    )
    return out
